# baseline (speedup 1.0000x reference)
import functools

import jax
import jax.numpy as jnp
from jax import lax
from jax.experimental import pallas as pl
from jax.experimental.pallas import tpu as pltpu
from jax.experimental.pallas import tpu_sc as plsc

N = 4096
D = 768
B = 2
K = 10
R = 5
BR = 256
NRB = N // BR

_NEG = -3e38
_BIG = 1 << 30


def _rowkey_body(q_ref, k_ref, key_ref, m_ref, den_ref):
    qb = q_ref[0]
    kb = k_ref[0]
    s = lax.dot_general(qb, kb, (((1,), (1,)), ((), ())),
                        preferred_element_type=jnp.float32)
    m = jnp.max(s, axis=1, keepdims=True)
    e = jnp.exp(s - m)
    denom = jnp.sum(e, axis=1)
    i = pl.program_id(1)
    row_ids = lax.broadcasted_iota(jnp.int32, (BR, N), 0)
    col_ids = lax.broadcasted_iota(jnp.int32, (BR, N), 1)
    dmask = col_ids == row_ids + i * BR
    dexp = jnp.sum(jnp.where(dmask, e, 0.0), axis=1)
    key_ref[0, 0, 0, :] = dexp / denom
    m_ref[0, 0, 0, :] = m[:, 0]
    den_ref[0, 0, 0, :] = denom


def _rowkey(q, k):
    outs = pl.pallas_call(
        _rowkey_body,
        grid=(B, NRB),
        in_specs=[
            pl.BlockSpec((1, BR, D), lambda b, i: (b, i, 0)),
            pl.BlockSpec((1, N, D), lambda b, i: (b, 0, 0)),
        ],
        out_specs=[pl.BlockSpec((1, 1, 1, BR), lambda b, i: (b, i, 0, 0))] * 3,
        out_shape=[jax.ShapeDtypeStruct((B, NRB, 1, BR), jnp.float32)] * 3,
    )(q, k)
    return tuple(o.reshape(B, N) for o in outs)


_CHUNK = N // 16


def _iota16():
    return lax.broadcasted_iota(jnp.int32, (16,), 0)


def _perm(v, sh):
    dnums = lax.GatherDimensionNumbers(
        offset_dims=(), collapsed_slice_dims=(0,), start_index_map=(0,))
    return lax.gather(v, (_iota16() ^ sh)[:, None], dnums, (1,),
                      mode=lax.GatherScatterMode.PROMISE_IN_BOUNDS)


def _allmax(v):
    for sh in (8, 4, 2, 1):
        v = jnp.maximum(v, _perm(v, sh))
    return v


def _allmin(v):
    for sh in (8, 4, 2, 1):
        v = jnp.minimum(v, _perm(v, sh))
    return v


def _find_topk(val_ref, idx_ref, nchunks):
    lane = _iota16()
    cand_val = jnp.full((16,), -1.0, jnp.float32)
    cand_idx = jnp.full((16,), _BIG, jnp.int32)
    chosen = []

    for t in range(K):
        def mx_body(i, cur, chosen=tuple(chosen)):
            v = val_ref[pl.ds(i * 16, 16)]
            ids = idx_ref[pl.ds(i * 16, 16)]
            for g in chosen:
                v = jnp.where(ids == g, _NEG, v)
            return jnp.maximum(cur, v)
        mx = _allmax(lax.fori_loop(0, nchunks, mx_body,
                                   jnp.full((16,), _NEG, jnp.float32)))

        def ix_body(i, cur, chosen=tuple(chosen)):
            v = val_ref[pl.ds(i * 16, 16)]
            ids = idx_ref[pl.ds(i * 16, 16)]
            for g in chosen:
                ids = jnp.where(ids == g, _BIG, ids)
            return jnp.minimum(cur, jnp.where(v == mx, ids, _BIG))
        gix = _allmin(lax.fori_loop(0, nchunks, ix_body,
                                    jnp.full((16,), _BIG, jnp.int32)))

        cand_val = jnp.where(lane == t, mx, cand_val)
        cand_idx = jnp.where(lane == t, gix, cand_idx)
        chosen.append(gix)

    return cand_val, cand_idx


def _sort10_asc(idx_vec):
    lane = _iota16()
    out = jnp.full((16,), _BIG, jnp.int32)
    chosen = []
    for t in range(K):
        cur = idx_vec
        for g in chosen:
            cur = jnp.where(cur == g, _BIG, cur)
        mn = _allmin(cur)
        out = jnp.where(lane == t, mn, out)
        chosen.append(mn)
    return out


def _sc_topk_gather(key, q2, k2):
    mesh = plsc.VectorSubcoreMesh(core_axis_name="c", subcore_axis_name="s")

    @functools.partial(
        pl.kernel, mesh=mesh,
        out_type=[
            jax.ShapeDtypeStruct((B, 16), jnp.int32),
            jax.ShapeDtypeStruct((B, 16, D), jnp.float32),
            jax.ShapeDtypeStruct((B, 16, D), jnp.float32),
        ],
        scratch_types=[
            pltpu.VMEM((_CHUNK,), jnp.float32),
            pltpu.VMEM((_CHUNK,), jnp.int32),
            pltpu.VMEM((16,), jnp.float32),
            pltpu.VMEM((16,), jnp.int32),
            pltpu.VMEM((256,), jnp.float32),
            pltpu.VMEM((256,), jnp.int32),
            pltpu.VMEM((16,), jnp.int32),
            pltpu.VMEM((16,), jnp.int32),
            pltpu.VMEM((16, D), jnp.float32),
            pltpu.VMEM((16, D), jnp.float32),
            pltpu.VMEM_SHARED((256,), jnp.float32),
            pltpu.VMEM_SHARED((256,), jnp.int32),
            pltpu.SemaphoreType.DMA,
            pltpu.SemaphoreType.DMA,
        ],
    )
    def sc_kernel(key_hbm, q_hbm, k_hbm,
                  tk_hbm, qtop_hbm, ktop_hbm,
                  keyv, kidx, cval, cidx, mval, midx, tidx, gidx,
                  qrows, krows, sh_val, sh_idx, sem_q, sem_k):
        b = lax.axis_index("c")
        s = lax.axis_index("s")
        base = pl.multiple_of(b * N + s * _CHUNK, 256)

        pltpu.sync_copy(key_hbm.at[pl.ds(base, _CHUNK)], keyv)
        gbase = s * _CHUNK
        for i in range(_CHUNK // 16):
            kidx[pl.ds(i * 16, 16)] = gbase + i * 16 + _iota16()
        lv, li = _find_topk(keyv, kidx, _CHUNK // 16)
        cval[...] = lv
        cidx[...] = li

        off = pl.multiple_of(s * 16, 16)
        pltpu.sync_copy(cval, sh_val.at[pl.ds(off, 16)])
        pltpu.sync_copy(cidx, sh_idx.at[pl.ds(off, 16)])
        plsc.subcore_barrier()

        @pl.when(s == 0)
        def _merge():
            pltpu.sync_copy(sh_val, mval)
            pltpu.sync_copy(sh_idx, midx)
            _, top_idx = _find_topk(mval, midx, 16)
            srt = _sort10_asc(top_idx)
            tidx[...] = srt
            pltpu.sync_copy(tidx, tk_hbm.at[b])

            clamped = jnp.minimum(srt, N - 1)
            gidx[...] = clamped + b * N
            cp_q = pltpu.async_copy(q_hbm.at[gidx], qrows, sem_q)
            cp_k = pltpu.async_copy(k_hbm.at[gidx], krows, sem_k)
            cp_q.wait()
            cp_k.wait()
            pltpu.sync_copy(qrows, qtop_hbm.at[b])
            pltpu.sync_copy(krows, ktop_hbm.at[b])

    return sc_kernel(key.reshape(B * N), q2, k2)


def _select_body(tk_ref, qt_ref, kt_ref, m_ref, den_ref, obj_refs, rel_refs):
    tvec = tk_ref[0]
    q_top = qt_ref[0]
    k_top = kt_ref[0]
    r_i = lax.broadcasted_iota(jnp.int32, (16, 16), 0)
    c_i = lax.broadcasted_iota(jnp.int32, (16, 16), 1)
    diag = r_i == c_i
    tk_bc = jnp.broadcast_to(tvec, (16, 16))
    tcol = jnp.sum(jnp.where(diag, tk_bc, 0), axis=1, keepdims=True)
    colN = lax.broadcasted_iota(jnp.int32, (16, N), 1)
    ohb_rows = colN == tcol
    m_col = jnp.sum(jnp.where(ohb_rows, jnp.broadcast_to(m_ref[0], (16, N)),
                              0.0), axis=1, keepdims=True)
    den_col = jnp.sum(jnp.where(ohb_rows, jnp.broadcast_to(den_ref[0], (16, N)),
                                0.0), axis=1, keepdims=True)

    s10 = lax.dot_general(q_top, k_top, (((1,), (1,)), ((), ())),
                          preferred_element_type=jnp.float32)
    rs10 = jnp.exp(s10 - m_col) / den_col
    valid = (r_i < K) & (c_i < K)
    rs10 = jnp.where(valid, rs10, _NEG)

    scur = rs10
    sel = jnp.zeros((16, 16), dtype=jnp.bool_)
    for _ in range(R):
        mx = jnp.max(scur, axis=1, keepdims=True)
        cj = jnp.min(jnp.where(scur == mx, c_i, _BIG), axis=1, keepdims=True)
        hit = c_i == cj
        sel = sel | hit
        scur = jnp.where(hit, _NEG, scur)

    selF = sel.astype(jnp.float32)
    lt = (r_i <= c_i).astype(jnp.float32)
    prefix = jnp.dot(selF, lt, preferred_element_type=jnp.float32)

    trow = jnp.broadcast_to(tvec, (16, 16))
    for j in range(R):
        ohb = sel & (prefix == (j + 1.0))
        ohf = ohb.astype(jnp.float32)
        objid = jnp.sum(jnp.where(ohb, trow, 0), axis=1)
        obj_refs[j][0, 0, :] = objid
        eobj = jnp.dot(ohf, q_top, preferred_element_type=jnp.float32)
        rel0 = q_top + eobj
        mean = jnp.mean(rel0, axis=1, keepdims=True)
        var = jnp.mean((rel0 - mean) ** 2, axis=1, keepdims=True)
        rel_refs[j][0] = (rel0 - mean) / jnp.sqrt(var + 1e-5)


def _select_wrap(tk_ref, qt_ref, kt_ref, m_ref, den_ref,
                 o0, o1, o2, o3, o4, e0, e1, e2, e3, e4):
    _select_body(tk_ref, qt_ref, kt_ref, m_ref, den_ref,
                 [o0, o1, o2, o3, o4], [e0, e1, e2, e3, e4])


def _select(tk, q_top, k_top, m, den):
    outs = pl.pallas_call(
        _select_wrap,
        grid=(B,),
        in_specs=[
            pl.BlockSpec((1, 1, 16), lambda b: (b, 0, 0)),
            pl.BlockSpec((1, 16, D), lambda b: (b, 0, 0)),
            pl.BlockSpec((1, 16, D), lambda b: (b, 0, 0)),
            pl.BlockSpec((1, 1, N), lambda b: (b, 0, 0)),
            pl.BlockSpec((1, 1, N), lambda b: (b, 0, 0)),
        ],
        out_specs=[pl.BlockSpec((1, 1, 16), lambda b: (b, 0, 0))] * R
        + [pl.BlockSpec((1, 16, D), lambda b: (b, 0, 0))] * R,
        out_shape=[jax.ShapeDtypeStruct((B, 1, 16), jnp.int32)] * R
        + [jax.ShapeDtypeStruct((B, 16, D), jnp.float32)] * R,
    )(tk.reshape(B, 1, 16), q_top, k_top,
      m.reshape(B, 1, N), den.reshape(B, 1, N))
    objs = [o[:, 0, :] for o in outs[:R]]
    rels = outs[R:]
    return objs, rels


def kernel(q, k, top_k_instances, top_k_relationships):
    del top_k_instances, top_k_relationships
    key, m, den = _rowkey(q, k)
    q2 = q.reshape(B * N, D)
    k2 = k.reshape(B * N, D)
    tk, q_top, k_top = _sc_topk_gather(key, q2, k2)
    objs, rels = _select(tk, q_top, k_top, m, den)
    obj50 = jnp.stack(objs, axis=-1)[:, :K, :].reshape(B, K * R)
    sub50 = jnp.repeat(tk[:, :K], R, axis=1)
    bids = jnp.broadcast_to(jnp.arange(B, dtype=jnp.int32)[:, None], (B, K * R))
    soi = jnp.stack([bids, sub50, obj50], axis=-1)
    rel = jnp.stack(rels, axis=2)[:, :K].reshape(B, K * R, D)
    return soi, rel
